# baseline (speedup 1.0000x reference)
import jax
import jax.numpy as jnp
from jax.experimental import pallas as pl
from jax.experimental.pallas import tpu as pltpu

_B = 128
_N = 8192
_K = 64
_OUT = 16
_TEMP = 0.5


def _gate_kernel(scores_ref, out_ref, gate_ref):
    @pl.when(pl.program_id(0) == 0)
    def _compute_gate():
        st = scores_ref[...].T
        bits = jax.lax.bitcast_convert_type(st, jnp.int32)

        lo = jnp.zeros((_OUT, 1), jnp.int32)
        hi = jnp.full((_OUT, 1), 0x7F800000, jnp.int32)

        def search_body(_, carry):
            lo, hi = carry
            mid = (lo + hi) >> 1
            cnt = jnp.sum((bits >= mid).astype(jnp.int32), axis=1,
                          keepdims=True)
            ge = cnt >= _K
            return jnp.where(ge, mid, lo), jnp.where(ge, hi, mid)

        lo, hi = jax.lax.fori_loop(0, 31, search_body, (lo, hi))
        tau = lo

        gt = bits > tau
        n_gt = jnp.sum(gt.astype(jnp.int32), axis=1, keepdims=True)
        need = _K - n_gt
        tie = bits == tau
        idx = jax.lax.broadcasted_iota(jnp.int32, (_OUT, _N), 1)

        lo2 = jnp.zeros((_OUT, 1), jnp.int32)
        hi2 = jnp.full((_OUT, 1), _N + 1, jnp.int32)

        def tie_body(_, carry):
            lo, hi = carry
            mid = (lo + hi) >> 1
            cnt = jnp.sum((tie & (idx < mid)).astype(jnp.int32), axis=1,
                          keepdims=True)
            ok = cnt <= need
            return jnp.where(ok, mid, lo), jnp.where(ok, hi, mid)

        lo2, hi2 = jax.lax.fori_loop(0, 14, tie_body, (lo2, hi2))
        keep = gt | (tie & (idx < lo2))

        m = jnp.max(st, axis=1, keepdims=True)
        e = jnp.where(keep, jnp.exp((st - m) / _TEMP), 0.0)
        gate_ref[...] = (e / jnp.sum(e, axis=1, keepdims=True)).T

    out_ref[...] = jnp.broadcast_to(gate_ref[...][None], out_ref.shape)


_BB = 4


def kernel(X, scores):
    del X
    return pl.pallas_call(
        _gate_kernel,
        grid=(_B // _BB,),
        in_specs=[pl.BlockSpec((_N, _OUT), lambda b: (0, 0))],
        out_specs=pl.BlockSpec((_BB, _N, _OUT), lambda b: (b, 0, 0)),
        out_shape=jax.ShapeDtypeStruct((_B, _N, _OUT), jnp.float32),
        scratch_shapes=[pltpu.VMEM((_N, _OUT), jnp.float32)],
    )(scores)


# speedup vs baseline: 2.7406x; 2.7406x over previous
import jax
import jax.numpy as jnp
from jax.experimental import pallas as pl
from jax.experimental.pallas import tpu as pltpu

_B = 128
_N = 8192
_K = 64
_OUT = 16
_TEMP = 0.5
_LIN = _N * _OUT // 128
_BB = 16


def _colsum(x):
    r = jnp.sum(x, axis=0, keepdims=True)
    for sh in (16, 32, 64):
        r = r + jnp.concatenate([r[:, sh:], r[:, :sh]], axis=1)
    return r


def _colmax(x):
    r = jnp.max(x, axis=0, keepdims=True)
    for sh in (16, 32, 64):
        r = jnp.maximum(r, jnp.concatenate([r[:, sh:], r[:, :sh]], axis=1))
    return r


def _gate_kernel(scores_ref, out_ref, gate_ref):
    @pl.when(pl.program_id(0) == 0)
    def _compute_gate():
        s = scores_ref[...]
        bits = jax.lax.bitcast_convert_type(s, jnp.int32)

        lo = jnp.zeros((1, 128), jnp.int32)
        hi = jnp.full((1, 128), 0x7F800000, jnp.int32)

        def search_body(_, carry):
            lo, hi = carry
            mid = (lo + hi) >> 1
            cnt = _colsum((bits >= mid).astype(jnp.int32))
            ge = cnt >= _K
            return jnp.where(ge, mid, lo), jnp.where(ge, hi, mid)

        lo, hi = jax.lax.fori_loop(0, 31, search_body, (lo, hi))
        tau = lo

        gt = bits > tau
        n_gt = _colsum(gt.astype(jnp.int32))
        need = _K - n_gt
        tie = bits == tau

        p_idx = jax.lax.broadcasted_iota(jnp.int32, (_LIN, 128), 0)
        q_idx = jax.lax.broadcasted_iota(jnp.int32, (_LIN, 128), 1)
        idx = (p_idx << 3) + (q_idx >> 4)

        lo2 = jnp.zeros((1, 128), jnp.int32)
        hi2 = jnp.full((1, 128), _N + 1, jnp.int32)

        def tie_body(_, carry):
            lo, hi = carry
            mid = (lo + hi) >> 1
            cnt = _colsum((tie & (idx < mid)).astype(jnp.int32))
            ok = cnt <= need
            return jnp.where(ok, mid, lo), jnp.where(ok, hi, mid)

        lo2, hi2 = jax.lax.fori_loop(0, 14, tie_body, (lo2, hi2))
        keep = gt | (tie & (idx < lo2))

        m = _colmax(s)
        e = jnp.where(keep, jnp.exp((s - m) / _TEMP), 0.0)
        gate_ref[...] = e / _colsum(e)

    out_ref[...] = jnp.broadcast_to(gate_ref[...][None], out_ref.shape)


def kernel(X, scores):
    del X
    flat = pl.pallas_call(
        _gate_kernel,
        grid=(_B // _BB,),
        in_specs=[pl.BlockSpec((_LIN, 128), lambda b: (0, 0))],
        out_specs=pl.BlockSpec((_BB, _LIN, 128), lambda b: (b, 0, 0)),
        out_shape=jax.ShapeDtypeStruct((_B, _LIN, 128), jnp.float32),
        scratch_shapes=[pltpu.VMEM((_LIN, 128), jnp.float32)],
    )(scores.reshape(_LIN, 128))
    return flat.reshape(_B, _N, _OUT)


# speedup vs baseline: 15.0213x; 5.4810x over previous
import jax
import jax.numpy as jnp
from jax.experimental import pallas as pl
from jax.experimental.pallas import tpu as pltpu

_B = 128
_N = 8192
_K = 64
_OUT = 16
_TEMP = 0.5
_BB = 16


def _gate_kernel(scores_ref, out_ref, gate_ref):
    @pl.when(pl.program_id(0) == 0)
    def _compute_gate():
        st = scores_ref[...]
        bits = jax.lax.bitcast_convert_type(st, jnp.int32)

        lo = jnp.zeros((_OUT, 1), jnp.int32)
        hi = jnp.full((_OUT, 1), 0x7F800000, jnp.int32)

        def search_body(_, carry):
            lo, hi = carry
            mid = (lo + hi) >> 1
            cnt = jnp.sum((bits >= mid).astype(jnp.int32), axis=1,
                          keepdims=True)
            ge = cnt >= _K
            return jnp.where(ge, mid, lo), jnp.where(ge, hi, mid)

        lo, hi = jax.lax.fori_loop(0, 31, search_body, (lo, hi))
        tau = lo

        gt = bits > tau
        n_gt = jnp.sum(gt.astype(jnp.int32), axis=1, keepdims=True)
        need = _K - n_gt
        tie = bits == tau
        idx = jax.lax.broadcasted_iota(jnp.int32, (_OUT, _N), 1)

        lo2 = jnp.zeros((_OUT, 1), jnp.int32)
        hi2 = jnp.full((_OUT, 1), _N + 1, jnp.int32)

        def tie_body(_, carry):
            lo, hi = carry
            mid = (lo + hi) >> 1
            cnt = jnp.sum((tie & (idx < mid)).astype(jnp.int32), axis=1,
                          keepdims=True)
            ok = cnt <= need
            return jnp.where(ok, mid, lo), jnp.where(ok, hi, mid)

        lo2, hi2 = jax.lax.fori_loop(0, 14, tie_body, (lo2, hi2))
        keep = gt | (tie & (idx < lo2))

        m = jnp.max(st, axis=1, keepdims=True)
        e = jnp.where(keep, jnp.exp((st - m) / _TEMP), 0.0)
        gate_ref[...] = e / jnp.sum(e, axis=1, keepdims=True)

    out_ref[...] = jnp.broadcast_to(gate_ref[...][None], out_ref.shape)


def kernel(X, scores):
    del X
    out_t = pl.pallas_call(
        _gate_kernel,
        grid=(_B // _BB,),
        in_specs=[pl.BlockSpec((_OUT, _N), lambda b: (0, 0))],
        out_specs=pl.BlockSpec((_BB, _OUT, _N), lambda b: (b, 0, 0)),
        out_shape=jax.ShapeDtypeStruct((_B, _OUT, _N), jnp.float32),
        scratch_shapes=[pltpu.VMEM((_OUT, _N), jnp.float32)],
    )(scores.T)
    return out_t.transpose(0, 2, 1)


# speedup vs baseline: 15.3157x; 1.0196x over previous
import jax
import jax.numpy as jnp
from jax.experimental import pallas as pl
from jax.experimental.pallas import tpu as pltpu

_B = 128
_N = 8192
_K = 64
_OUT = 16
_TEMP = 0.5
_BB = 16


def _gate_kernel(scores_ref, out_ref, gate_ref, sem):
    st = scores_ref[...]
    bits = jax.lax.bitcast_convert_type(st, jnp.int32)

    lo = jnp.zeros((_OUT, 1), jnp.int32)
    hi = jnp.full((_OUT, 1), 0x7F800000, jnp.int32)

    def search_body(_, carry):
        lo, hi = carry
        mid = (lo + hi) >> 1
        cnt = jnp.sum((bits >= mid).astype(jnp.int32), axis=1,
                      keepdims=True)
        ge = cnt >= _K
        return jnp.where(ge, mid, lo), jnp.where(ge, hi, mid)

    lo, hi = jax.lax.fori_loop(0, 31, search_body, (lo, hi))
    tau = lo

    gt = bits > tau
    n_gt = jnp.sum(gt.astype(jnp.int32), axis=1, keepdims=True)
    need = _K - n_gt
    tie = bits == tau
    idx = jax.lax.broadcasted_iota(jnp.int32, (_OUT, _N), 1)

    lo2 = jnp.zeros((_OUT, 1), jnp.int32)
    hi2 = jnp.full((_OUT, 1), _N + 1, jnp.int32)

    def tie_body(_, carry):
        lo, hi = carry
        mid = (lo + hi) >> 1
        cnt = jnp.sum((tie & (idx < mid)).astype(jnp.int32), axis=1,
                      keepdims=True)
        ok = cnt <= need
        return jnp.where(ok, mid, lo), jnp.where(ok, hi, mid)

    lo2, hi2 = jax.lax.fori_loop(0, 14, tie_body, (lo2, hi2))
    keep = gt | (tie & (idx < lo2))

    m = jnp.max(st, axis=1, keepdims=True)
    e = jnp.where(keep, jnp.exp((st - m) / _TEMP), 0.0)
    gate_ref[...] = e / jnp.sum(e, axis=1, keepdims=True)

    def _start(b, _):
        pltpu.make_async_copy(gate_ref, out_ref.at[b], sem).start()
        return 0

    jax.lax.fori_loop(0, _B, _start, 0)

    def _wait(b, _):
        pltpu.make_async_copy(gate_ref, out_ref.at[b], sem).wait()
        return 0

    jax.lax.fori_loop(0, _B, _wait, 0)


def kernel(X, scores):
    del X
    out_t = pl.pallas_call(
        _gate_kernel,
        in_specs=[pl.BlockSpec(memory_space=pltpu.VMEM)],
        out_specs=pl.BlockSpec(memory_space=pl.ANY),
        out_shape=jax.ShapeDtypeStruct((_B, _OUT, _N), jnp.float32),
        scratch_shapes=[pltpu.VMEM((_OUT, _N), jnp.float32),
                        pltpu.SemaphoreType.DMA],
    )(scores.T)
    return out_t.transpose(0, 2, 1)
